# baseline (speedup 1.0000x reference)
import functools

import jax
import jax.numpy as jnp
from jax import lax
from jax.experimental import pallas as pl
from jax.experimental.pallas import tpu as pltpu
from jax.experimental.pallas import tpu_sc as plsc

B = 16384
ZIP_V = 100000
NC = 2
NS = 16
NW = NC * NS
B_PER_W = B // NW
CH = 128
N_CHUNKS = B_PER_W // CH

BT = 1024
BN_EPS = 1e-3


@functools.lru_cache(maxsize=1)
def _make_sc_gather():
    mesh = plsc.VectorSubcoreMesh(core_axis_name="c", subcore_axis_name="s")

    @functools.partial(
        pl.kernel,
        mesh=mesh,
        out_type=[
            jax.ShapeDtypeStruct((B, 128), jnp.float32),
            jax.ShapeDtypeStruct((B * 64,), jnp.float32),
        ],
        scratch_types=[
            pltpu.VMEM((B_PER_W,), jnp.int32),
            pltpu.VMEM((B_PER_W,), jnp.int32),
            pltpu.VMEM((2, CH, 128), jnp.float32),
            pltpu.VMEM((B_PER_W * 64,), jnp.float32),
            pltpu.SemaphoreType.DMA,
            pltpu.SemaphoreType.DMA,
        ],
    )
    def sc_gather(uid_hbm, zid_hbm, utab_hbm, ztab_hbm, out_u, out_z,
                  uidx_v, zidx_v, urows_v, zrows_v, usem, zsem):
        wid = lax.axis_index("s") * NC + lax.axis_index("c")
        base = wid * B_PER_W
        pltpu.sync_copy(uid_hbm.at[pl.ds(base, B_PER_W)], uidx_v)
        pltpu.sync_copy(zid_hbm.at[pl.ds(base, B_PER_W)], zidx_v)

        def zfire(g, carry):
            o = g * 16
            vec = zidx_v[pl.ds(o, 16)]
            for lane in range(16):
                pltpu.async_copy(ztab_hbm.at[pl.ds(vec[lane] * 64, 64)],
                                 zrows_v.at[pl.ds((o + lane) * 64, 64)], zsem)
            return carry
        lax.fori_loop(0, B_PER_W // 16, zfire, 0)

        def fire(ci):
            o = ci * CH
            b = ci % 2
            return pltpu.async_copy(utab_hbm.at[uidx_v.at[pl.ds(o, CH)]],
                                    urows_v.at[b], usem)

        inflight = [fire(0), fire(1)]
        for ci in range(N_CHUNKS):
            b = ci % 2
            inflight[b].wait()
            pltpu.sync_copy(urows_v.at[b], out_u.at[pl.ds(base + ci * CH, CH)])
            if ci + 2 < N_CHUNKS:
                inflight[b] = fire(ci + 2)

        pltpu.make_async_copy(ztab_hbm.at[pl.ds(0, B_PER_W * 64)],
                              zrows_v, zsem).wait()
        pltpu.sync_copy(zrows_v, out_z.at[pl.ds(base * 64, B_PER_W * 64)])

    return sc_gather


def _mlp_body(ipack, fpack, ue, ze, w1u, w1z, wsm, ctab, w1c, wsc,
              vp1, vp2, w2, out):
    g = ipack[0, :]
    occ = ipack[1, :]
    age = ipack[2, :]
    dow = ipack[3, :]
    hod = ipack[4, :]
    cols = lax.broadcasted_iota(jnp.int32, (BT, 64), 1)
    m = ((cols == g[:, None])
         | (cols == occ[:, None] + 2)
         | (cols == age[:, None] + 24)
         | (cols == dow[:, None] + 31)
         | (cols == hod[:, None] + 38)
         | (cols == 62)).astype(jnp.float32)
    cross = (dow * 24 + hod) % 34 + 1
    cols40 = lax.broadcasted_iota(jnp.int32, (BT, 40), 1)
    mc = (cols40 == cross[:, None]).astype(jnp.float32)
    cproj = jnp.dot(ctab[...], w1c[...], preferred_element_type=jnp.float32)
    h = (jnp.dot(ue[...], w1u[...], preferred_element_type=jnp.float32)
         + jnp.dot(ze[...], w1z[...], preferred_element_type=jnp.float32)
         + jnp.dot(m, wsm[...], preferred_element_type=jnp.float32)
         + jnp.dot(mc, cproj, preferred_element_type=jnp.float32)
         + fpack[0, :][:, None] * wsc[0, :][None, :]
         + fpack[1, :][:, None] * wsc[1, :][None, :]
         + fpack[2, :][:, None] * wsc[2, :][None, :])
    h = jnp.maximum(h, 0.0)
    s1 = vp1[0, :] * lax.rsqrt(vp1[3, :] + BN_EPS)
    t1 = vp1[1, :] - vp1[2, :] * s1
    h = h * s1[None, :] + t1[None, :]
    h2 = jnp.dot(h, w2[...], preferred_element_type=jnp.float32) + vp2[0, :][None, :]
    h2 = jnp.maximum(h2, 0.0)
    s2 = vp2[1, :] * lax.rsqrt(vp2[4, :] + BN_EPS)
    t2 = vp2[2, :] - vp2[3, :] * s2
    h2 = h2 * s2[None, :] + t2[None, :]
    ss = jnp.sum(h2 * h2, axis=-1, keepdims=True)
    out[...] = h2 * lax.rsqrt(jnp.maximum(ss, 1e-12))


def _mlp_call(ipack, fpack, ue, ze, w1u, w1z, wsm, ctab, w1c, wsc,
              vp1, vp2, w2):
    grid = B // BT
    full = lambda shp: pl.BlockSpec(shp, lambda i: (0, 0))
    return pl.pallas_call(
        _mlp_body,
        grid=(grid,),
        in_specs=[
            pl.BlockSpec((5, BT), lambda i: (0, i)),
            pl.BlockSpec((3, BT), lambda i: (0, i)),
            pl.BlockSpec((BT, 128), lambda i: (i, 0)),
            pl.BlockSpec((BT, 64), lambda i: (i, 0)),
            full((128, 128)),
            full((64, 128)),
            full((64, 128)),
            full((40, 32)),
            full((32, 128)),
            full((3, 128)),
            full((4, 128)),
            full((5, 64)),
            full((128, 64)),
        ],
        out_specs=pl.BlockSpec((BT, 64), lambda i: (i, 0)),
        out_shape=jax.ShapeDtypeStruct((B, 64), jnp.float32),
    )(ipack, fpack, ue, ze, w1u, w1z, wsm, ctab, w1c, wsc, vp1, vp2, w2)


def kernel(user_gender, user_id, user_occupation_label, user_zip_code,
           bucketized_user_age, day_of_week, hour_of_day,
           example_age, example_age_square, example_age_sqrt,
           user_table, zip_table, cross_table,
           W1, b1, bn1_gamma, bn1_beta, bn1_mean, bn1_var,
           W2, b2, bn2_gamma, bn2_beta, bn2_mean, bn2_var):
    uid = user_id.astype(jnp.int32)
    zid = user_zip_code.astype(jnp.int32)
    ue, zeflat = _make_sc_gather()(uid, zid, user_table, zip_table.reshape(-1))
    ze = zeflat.reshape(B, 64)

    ipack = jnp.stack([
        user_gender.astype(jnp.int32),
        user_occupation_label.astype(jnp.int32),
        bucketized_user_age.astype(jnp.int32),
        day_of_week.astype(jnp.int32),
        hour_of_day.astype(jnp.int32),
    ])
    fpack = jnp.stack([example_age, example_age_square, example_age_sqrt])
    wsm = jnp.concatenate([
        W1[0:2], W1[130:152], W1[216:223], W1[223:230], W1[230:254],
        b1[None, :], jnp.zeros((1, 128), jnp.float32)], axis=0)
    ctab = jnp.concatenate([cross_table, jnp.zeros((5, 32), jnp.float32)], axis=0)
    vp1 = jnp.stack([bn1_gamma, bn1_beta, bn1_mean, bn1_var])
    vp2 = jnp.stack([b2, bn2_gamma, bn2_beta, bn2_mean, bn2_var])
    return _mlp_call(ipack, fpack, ue, ze,
                     W1[2:130], W1[152:216], wsm, ctab, W1[254:286],
                     W1[286:289], vp1, vp2, W2)


# speedup vs baseline: 2.0369x; 2.0369x over previous
import functools

import jax
import jax.numpy as jnp
from jax import lax
from jax.experimental import pallas as pl
from jax.experimental.pallas import tpu as pltpu
from jax.experimental.pallas import tpu_sc as plsc

B = 16384
ZIP_V = 100000
ZIP_D = 64
NC = 2
NS = 16
NW = NC * NS
B_PER_W = B // NW
CH = 64
N_CHUNKS = B_PER_W // CH
F_PER_W = ZIP_D // NW
ZQ = 2048
NZQ = B // ZQ

BT = 4096
BN_EPS = 1e-3


@functools.lru_cache(maxsize=1)
def _make_sc_gather():
    mesh = plsc.VectorSubcoreMesh(core_axis_name="c", subcore_axis_name="s")

    @functools.partial(
        pl.kernel,
        mesh=mesh,
        compiler_params=pltpu.CompilerParams(needs_layout_passes=False),
        out_type=[
            jax.ShapeDtypeStruct((B, 128), jnp.float32),
            jax.ShapeDtypeStruct((ZIP_D, B), jnp.float32),
        ],
        scratch_types=[
            pltpu.VMEM((B_PER_W,), jnp.int32),
            pltpu.VMEM((ZQ,), jnp.int32),
            pltpu.VMEM((ZQ,), jnp.int32),
            pltpu.VMEM((ZIP_V,), jnp.float32),
            pltpu.VMEM((ZQ,), jnp.float32),
            pltpu.VMEM((ZQ,), jnp.float32),
            pltpu.VMEM((2, CH, 128), jnp.float32),
            pltpu.SemaphoreType.DMA,
            pltpu.SemaphoreType.DMA,
            pltpu.SemaphoreType.DMA,
            pltpu.SemaphoreType.DMA,
        ],
    )
    def sc_gather(uid_hbm, zid_hbm, utab_hbm, ztabt_hbm, out_u, out_zt,
                  uidx_v, zidq0, zidq1, rowbuf_v, zoutq0, zoutq1, urows_v,
                  usem, zsem, zqsem, zosem):
        wid = lax.axis_index("s") * NC + lax.axis_index("c")
        base = wid * B_PER_W
        pltpu.sync_copy(uid_hbm.at[pl.ds(base, B_PER_W)], uidx_v)
        zrow_cp = pltpu.async_copy(ztabt_hbm.at[wid * F_PER_W], rowbuf_v,
                                   zsem)

        def fire(ci):
            o = ci * CH
            b = ci % 2
            return pltpu.async_copy(utab_hbm.at[uidx_v.at[pl.ds(o, CH)]],
                                    urows_v.at[b], usem)

        zidq = [zidq0, zidq1]
        zoutq = [zoutq0, zoutq1]
        idx_cp = [
            pltpu.async_copy(zid_hbm.at[pl.ds(0, ZQ)], zidq0, zqsem),
            pltpu.async_copy(zid_hbm.at[pl.ds(ZQ, ZQ)], zidq1, zqsem),
        ]
        out_cp = [None, None]
        n_tasks = F_PER_W * NZQ

        def do_zip_feature(f):
            c = wid * F_PER_W + f
            for q in range(NZQ):
                t = f * NZQ + q
                b = t % 2
                idx_cp[b].wait()
                if out_cp[b] is not None:
                    out_cp[b].wait()
                src = zidq[b]
                dst = zoutq[b]

                @plsc.parallel_loop(0, ZQ // 16, unroll=16)
                def _gq(g, _src=src, _dst=dst):
                    o = g * 16
                    _dst[pl.ds(o, 16)] = plsc.load_gather(
                        rowbuf_v, [_src[pl.ds(o, 16)]])

                out_cp[b] = pltpu.async_copy(
                    dst, out_zt.at[c, pl.ds(q * ZQ, ZQ)], zosem)
                nt = t + 2
                if nt < n_tasks:
                    nq = nt % NZQ
                    idx_cp[b] = pltpu.async_copy(
                        zid_hbm.at[pl.ds(nq * ZQ, ZQ)], zidq[b], zqsem)

        def user_chunks(lo, hi):
            for ci in range(lo, hi):
                b = ci % 2
                inflight[b].wait()
                pltpu.sync_copy(urows_v.at[b],
                                out_u.at[pl.ds(base + ci * CH, CH)])
                if ci + 2 < N_CHUNKS:
                    inflight[b] = fire(ci + 2)

        inflight = [fire(0), fire(1)]
        user_chunks(0, N_CHUNKS // 2)
        zrow_cp.wait()
        do_zip_feature(0)
        zrow_cp2 = pltpu.async_copy(
            ztabt_hbm.at[wid * F_PER_W + 1], rowbuf_v, zsem)
        user_chunks(N_CHUNKS // 2, N_CHUNKS)
        zrow_cp2.wait()
        do_zip_feature(1)
        for cp in out_cp:
            if cp is not None:
                cp.wait()

    return sc_gather


def _mlp_body(ipack, fpack, ue, zet, w1u, w1z, wsm, ctab, w1c, wsc,
              vp1, vp2, w2, out):
    g = ipack[0, :]
    occ = ipack[1, :]
    age = ipack[2, :]
    dow = ipack[3, :]
    hod = ipack[4, :]
    tdot = lambda a, b: lax.dot_general(a, b, (((0,), (0,)), ((), ())),
                                        preferred_element_type=jnp.float32)
    rows64 = lax.broadcasted_iota(jnp.int32, (64, BT), 0)
    mt = ((rows64 == g[None, :])
          | (rows64 == occ[None, :] + 2)
          | (rows64 == age[None, :] + 24)
          | (rows64 == dow[None, :] + 31)
          | (rows64 == hod[None, :] + 38)
          | (rows64 == 62)).astype(jnp.float32)
    cross = (dow * 24 + hod) % 34 + 1
    rows40 = lax.broadcasted_iota(jnp.int32, (40, BT), 0)
    mct = (rows40 == cross[None, :]).astype(jnp.float32)
    cproj = jnp.dot(ctab[...], w1c[...], preferred_element_type=jnp.float32)
    h = (jnp.dot(ue[...], w1u[...], preferred_element_type=jnp.float32)
         + tdot(zet[...], w1z[...])
         + tdot(mt, wsm[...])
         + tdot(mct, cproj)
         + tdot(fpack[...], wsc[...]))
    h = jnp.maximum(h, 0.0)
    s1 = vp1[0, :] * lax.rsqrt(vp1[3, :] + BN_EPS)
    t1 = vp1[1, :] - vp1[2, :] * s1
    h = h * s1[None, :] + t1[None, :]
    h2 = jnp.dot(h, w2[...], preferred_element_type=jnp.float32) + vp2[0, :][None, :]
    h2 = jnp.maximum(h2, 0.0)
    s2 = vp2[1, :] * lax.rsqrt(vp2[4, :] + BN_EPS)
    t2 = vp2[2, :] - vp2[3, :] * s2
    h2 = h2 * s2[None, :] + t2[None, :]
    ss = jnp.sum(h2 * h2, axis=-1, keepdims=True)
    out[...] = (h2 * lax.rsqrt(jnp.maximum(ss, 1e-12))).T


def _mlp_call(ipack, fpack, ue, zet, w1u, w1z, wsm, ctab, w1c, wsc,
              vp1, vp2, w2):
    grid = B // BT
    full = lambda shp: pl.BlockSpec(shp, lambda i: (0, 0))
    return pl.pallas_call(
        _mlp_body,
        grid=(grid,),
        in_specs=[
            pl.BlockSpec((5, BT), lambda i: (0, i)),
            pl.BlockSpec((3, BT), lambda i: (0, i)),
            pl.BlockSpec((BT, 128), lambda i: (i, 0)),
            pl.BlockSpec((64, BT), lambda i: (0, i)),
            full((128, 128)),
            full((64, 128)),
            full((64, 128)),
            full((40, 32)),
            full((32, 128)),
            full((3, 128)),
            full((4, 128)),
            full((5, 64)),
            full((128, 64)),
        ],
        out_specs=pl.BlockSpec((64, BT), lambda i: (0, i)),
        out_shape=jax.ShapeDtypeStruct((64, B), jnp.float32),
    )(ipack, fpack, ue, zet, w1u, w1z, wsm, ctab, w1c, wsc, vp1, vp2, w2)


def kernel(user_gender, user_id, user_occupation_label, user_zip_code,
           bucketized_user_age, day_of_week, hour_of_day,
           example_age, example_age_square, example_age_sqrt,
           user_table, zip_table, cross_table,
           W1, b1, bn1_gamma, bn1_beta, bn1_mean, bn1_var,
           W2, b2, bn2_gamma, bn2_beta, bn2_mean, bn2_var):
    uid = user_id.astype(jnp.int32)
    zid = user_zip_code.astype(jnp.int32)
    ue, zet = _make_sc_gather()(uid, zid, user_table, zip_table.T)

    ipack = jnp.stack([
        user_gender.astype(jnp.int32),
        user_occupation_label.astype(jnp.int32),
        bucketized_user_age.astype(jnp.int32),
        day_of_week.astype(jnp.int32),
        hour_of_day.astype(jnp.int32),
    ])
    fpack = jnp.stack([example_age, example_age_square, example_age_sqrt])
    wsm = jnp.concatenate([
        W1[0:2], W1[130:152], W1[216:223], W1[223:230], W1[230:254],
        b1[None, :], jnp.zeros((1, 128), jnp.float32)], axis=0)
    ctab = jnp.concatenate([cross_table, jnp.zeros((5, 32), jnp.float32)], axis=0)
    vp1 = jnp.stack([bn1_gamma, bn1_beta, bn1_mean, bn1_var])
    vp2 = jnp.stack([b2, bn2_gamma, bn2_beta, bn2_mean, bn2_var])
    out_t = _mlp_call(ipack, fpack, ue, zet,
                      W1[2:130], W1[152:216], wsm, ctab, W1[254:286],
                      W1[286:289], vp1, vp2, W2)
    return out_t.T
